# baseline (speedup 1.0000x reference)
import functools

import jax
import jax.numpy as jnp
from jax import lax
from jax.experimental import pallas as pl
from jax.experimental.pallas import tpu as pltpu
from jax.experimental.pallas import tpu_sc as plsc

NUM_CORES = 2
NUM_SUBCORES = 16
NUM_WORKERS = NUM_CORES * NUM_SUBCORES
EDGE_BATCH = 128

_SC_PARAMS = pltpu.CompilerParams(use_tc_tiling_on_sc=False)


def _row_chunks(n):
    for nz in range(NUM_SUBCORES, 0, -1):
        if n % nz == 0 and (n // nz) % 8 == 0:
            return nz, n // nz
    return 1, n


def _sc_degree(src2d, dst2d, n):
    nbt, b = src2d.shape
    nbw = nbt // NUM_WORKERS
    nz, zch = _row_chunks(n)
    mesh = plsc.VectorSubcoreMesh(core_axis_name="c", subcore_axis_name="s")

    @functools.partial(
        pl.kernel,
        out_type=jax.ShapeDtypeStruct((2, NUM_CORES, n, 16), jnp.float32),
        mesh=mesh,
        scratch_types=[
            pltpu.VMEM((nbw, b), jnp.int32),
            pltpu.VMEM((nbw, b), jnp.int32),
            pltpu.VMEM((b, 16), jnp.float32),
            pltpu.VMEM((zch, 16), jnp.float32),
            pltpu.VMEM_SHARED((n + 8, 16), jnp.float32),
            pltpu.VMEM_SHARED((n + 8, 16), jnp.float32),
            pltpu.SemaphoreType.DMA,
        ],
        compiler_params=_SC_PARAMS,
    )
    def deg_kernel(src_hbm, dst_hbm, out_hbm,
                   sidx, didx, ones_v, zbuf, dsrc_sh, ddst_sh, sem):
        c = lax.axis_index("c")
        s = lax.axis_index("s")
        wid = c * NUM_SUBCORES + s
        r0 = s * zch
        pltpu.sync_copy(src_hbm.at[pl.ds(wid * nbw, nbw)], sidx)
        pltpu.sync_copy(dst_hbm.at[pl.ds(wid * nbw, nbw)], didx)

        @pl.loop(0, b)
        def _(i):
            ones_v[i] = jnp.full((16,), 1.0, jnp.float32)

        @pl.when(s < nz)
        def _():
            @pl.loop(0, zch)
            def _(i):
                zbuf[i] = jnp.zeros((16,), jnp.float32)

            pltpu.sync_copy(zbuf, dsrc_sh.at[pl.ds(r0, zch)])
            pltpu.sync_copy(zbuf, ddst_sh.at[pl.ds(r0, zch)])

        plsc.subcore_barrier()

        @pl.loop(0, nbw)
        def _(i):
            pltpu.async_copy(ones_v, dsrc_sh.at[sidx.at[i]], sem, add=True)
            pltpu.async_copy(ones_v, ddst_sh.at[didx.at[i]], sem, add=True)

        @pl.loop(0, 2 * nbw)
        def _(i):
            pltpu.make_async_copy(ones_v, dsrc_sh.at[sidx.at[0]], sem).wait()

        plsc.subcore_barrier()

        @pl.when(s < nz)
        def _():
            pltpu.sync_copy(dsrc_sh.at[pl.ds(r0, zch)],
                            out_hbm.at[0, c, pl.ds(r0, zch)])
            pltpu.sync_copy(ddst_sh.at[pl.ds(r0, zch)],
                            out_hbm.at[1, c, pl.ds(r0, zch)])

    return deg_kernel(src2d, dst2d)


def _sc_mega(u0, rio, src2d, dst2d, n, cp):
    nbt, b = src2d.shape
    nbw = nbt // NUM_WORKERS
    nz, zch = _row_chunks(n)
    mch = 200 if zch % 200 == 0 else zch
    ncnk = zch // mch
    mesh = plsc.VectorSubcoreMesh(core_axis_name="c", subcore_axis_name="s")
    pshape = jax.ShapeDtypeStruct((NUM_CORES, n, cp), jnp.float32)

    @functools.partial(
        pl.kernel,
        out_type=[pshape, pshape, pshape,
                  jax.ShapeDtypeStruct((NUM_CORES, n + 8, cp), jnp.float32)],
        mesh=mesh,
        scratch_types=[
            pltpu.VMEM((nbw, b), jnp.int32),
            pltpu.VMEM((nbw, b), jnp.int32),
            pltpu.VMEM((b, cp), jnp.float32),
            pltpu.VMEM((b, cp), jnp.float32),
            pltpu.VMEM((b, cp), jnp.float32),
            pltpu.VMEM((b, cp), jnp.float32),
            pltpu.VMEM((mch, cp), jnp.float32),
            pltpu.VMEM((mch, cp), jnp.float32),
            pltpu.VMEM((mch, cp), jnp.float32),
            pltpu.VMEM((mch, cp), jnp.float32),
            pltpu.VMEM((mch, cp), jnp.float32),
            pltpu.VMEM_SHARED((n + 8, cp), jnp.float32),
            pltpu.SemaphoreType.DMA,
            pltpu.SemaphoreType.DMA,
            pltpu.SemaphoreType.DMA,
            pltpu.SemaphoreType.DMA,
            pltpu.SemaphoreType.DMA,
            pltpu.SemaphoreType.DMA,
            pltpu.SemaphoreType.DMA,
            pltpu.SemaphoreType.DMA,
            pltpu.SemaphoreType.DMA,
            pltpu.SemaphoreType.DMA,
            pltpu.SemaphoreType.REGULAR,
        ],
        compiler_params=_SC_PARAMS,
    )
    def mega_kernel(u0_hbm, rio_hbm, src_hbm, dst_hbm,
                    p1_hbm, p2_hbm, p3_hbm, u_scr,
                    sidx, didx, r0b, r1b, r2b, r3b,
                    i0, i1, i2, uca, ucb, acc_sh,
                    g0, g1, g2, g3, s0, s1, s2, s3, isem, osem, xsem):
        rows = (r0b, r1b, r2b, r3b)
        gsem = (g0, g1, g2, g3)
        ssem = (s0, s1, s2, s3)
        ucnks = (uca, ucb)
        c = lax.axis_index("c")
        s = lax.axis_index("s")
        wid = c * NUM_SUBCORES + s
        r0z = s * zch

        pltpu.sync_copy(src_hbm.at[pl.ds(wid * nbw, nbw)], sidx)
        pltpu.sync_copy(dst_hbm.at[pl.ds(wid * nbw, nbw)], didx)

        def zero_acc():
            @pl.when(s < nz)
            def _():
                @pl.loop(0, mch)
                def _(i):
                    for q in range(cp // 16):
                        i0[i, pl.ds(q * 16, 16)] = jnp.zeros(
                            (16,), jnp.float32)

                for t in range(ncnk):
                    pltpu.sync_copy(i0, acc_sh.at[pl.ds(r0z + t * mch, mch)])

        def sparse_phase(u_ref):
            @pl.loop(0, nbw + 4, step=4)
            def _(v):
                for j in range(4):
                    i = v + j

                    @pl.when((i >= 4) & (i < nbw))
                    def _():
                        pltpu.make_async_copy(
                            rows[j], acc_sh.at[didx.at[0]], ssem[j]).wait()

                    @pl.when(i < nbw)
                    def _():
                        pltpu.async_copy(u_ref.at[sidx.at[i]], rows[j],
                                         gsem[j])

                    k2 = i - 2
                    jb = (j + 2) % 4

                    @pl.when((k2 >= 0) & (k2 < nbw))
                    def _():
                        pltpu.make_async_copy(
                            u_ref.at[sidx.at[0]], rows[jb], gsem[jb]).wait()
                        pltpu.async_copy(rows[jb], acc_sh.at[didx.at[k2]],
                                         ssem[jb], add=True)

            for j in range(4):
                pltpu.make_async_copy(rows[j], acc_sh.at[didx.at[0]],
                                      ssem[j]).wait()

        def writeback(p_hbm):
            @pl.when(s < nz)
            def _():
                pltpu.sync_copy(acc_sh.at[pl.ds(r0z, zch)],
                                p_hbm.at[c, pl.ds(r0z, zch)])

        def xsync():
            plsc.subcore_barrier()
            pl.semaphore_signal(xsem, 1, core_index=1 - c)
            pl.semaphore_wait(xsem, 1)

        def combine(p_ref):
            @pl.when(s < nz)
            def _():
                def fetch(t):
                    rr = r0z + t * mch
                    pltpu.async_copy(p_ref.at[0, pl.ds(rr, mch)], i0, isem)
                    pltpu.async_copy(p_ref.at[1, pl.ds(rr, mch)], i1, isem)
                    pltpu.async_copy(rio_hbm.at[pl.ds(rr, mch)], i2, isem)

                fetch(0)
                for t in range(ncnk):
                    rr = r0z + t * mch
                    uc = ucnks[t % 2]
                    for _ in range(3):
                        pltpu.make_async_copy(
                            rio_hbm.at[pl.ds(rr, mch)], i2, isem).wait()
                    if t >= 2:
                        pltpu.make_async_copy(
                            uc, u_scr.at[c, pl.ds(rr, mch)], osem).wait()

                    @pl.loop(0, mch, step=8)
                    def _(r):
                        for dr in range(8):
                            for q in range(cp // 16):
                                sl = (r + dr, pl.ds(q * 16, 16))
                                uc[sl] = i2[sl] * (i0[sl] + i1[sl])

                    if t + 1 < ncnk:
                        fetch(t + 1)
                    pltpu.async_copy(uc, u_scr.at[c, pl.ds(rr, mch)], osem)

                for _ in range(min(2, ncnk)):
                    pltpu.make_async_copy(
                        uca, u_scr.at[c, pl.ds(r0z, mch)], osem).wait()

        zero_acc()
        plsc.subcore_barrier()
        sparse_phase(u0_hbm)
        plsc.subcore_barrier()
        writeback(p1_hbm)
        zero_acc()
        xsync()

        combine(p1_hbm)
        plsc.subcore_barrier()
        sparse_phase(u_scr.at[c])
        plsc.subcore_barrier()
        writeback(p2_hbm)
        zero_acc()
        xsync()

        combine(p2_hbm)
        plsc.subcore_barrier()
        sparse_phase(u_scr.at[c])
        plsc.subcore_barrier()
        writeback(p3_hbm)

    res = mega_kernel(u0, rio, src2d, dst2d)
    return res[0], res[1], res[2]


def _tc_project(feature, w_padded, n, cp):
    blk = 1000

    def body(f_ref, w_ref, o_ref):
        o_ref[...] = jnp.dot(f_ref[...], w_ref[...],
                             preferred_element_type=jnp.float32)

    d = feature.shape[1]
    return pl.pallas_call(
        body,
        grid=(n // blk,),
        in_specs=[pl.BlockSpec((blk, d), lambda i: (i, 0)),
                  pl.BlockSpec((d, cp), lambda i: (0, 0))],
        out_specs=pl.BlockSpec((blk, cp), lambda i: (i, 0)),
        out_shape=jax.ShapeDtypeStruct((n, cp), jnp.float32),
    )(feature, w_padded)


def _tc_prep(y0, degs, n, cp):
    def body(y_ref, d_ref, u_ref, rio_ref, rin_ref):
        deg_out = d_ref[0, 0, :, 0:1] + d_ref[0, 1, :, 0:1]
        deg_in = d_ref[1, 0, :, 0:1] + d_ref[1, 1, :, 0:1]
        rout = lax.rsqrt(jnp.maximum(deg_out, 1.0))
        rin = lax.rsqrt(jnp.maximum(deg_in, 1.0))
        u_ref[...] = jnp.concatenate(
            [y_ref[...] * rout, jnp.zeros((8, cp), jnp.float32)], axis=0)
        rio_ref[...] = jnp.broadcast_to(rin * rout, (n, cp))
        rin_ref[...] = rin

    return pl.pallas_call(
        body,
        out_shape=[
            jax.ShapeDtypeStruct((n + 8, cp), jnp.float32),
            jax.ShapeDtypeStruct((n, cp), jnp.float32),
            jax.ShapeDtypeStruct((n, 1), jnp.float32),
        ],
    )(y0, degs)


def _tc_final(p1, p2, p3, y0, rin, b2d, n, c_out):
    def body(p1_ref, p2_ref, p3_ref, y_ref, rin_ref, b_ref, o_ref):
        psum = ((p1_ref[0] + p1_ref[1]) + (p2_ref[0] + p2_ref[1])
                + (p3_ref[0] + p3_ref[1]))
        res = (y_ref[...] + psum * rin_ref[...]) * 0.25
        o_ref[...] = res[:, :c_out] + b_ref[...]

    return pl.pallas_call(
        body, out_shape=jax.ShapeDtypeStruct((n, c_out), jnp.float32),
    )(p1, p2, p3, y0, rin, b2d)


def kernel(feature, edge_index, W, b):
    n, d = feature.shape
    c_out = W.shape[1]
    e = edge_index.shape[1]
    cp = ((c_out + 15) // 16) * 16

    nbt = -(-e // (EDGE_BATCH * NUM_WORKERS)) * NUM_WORKERS
    epad = nbt * EDGE_BATCH - e
    trash = jnp.full((epad,), n, jnp.int32)
    src2d = jnp.concatenate([edge_index[0], trash]).reshape(nbt, EDGE_BATCH)
    dst2d = jnp.concatenate([edge_index[1], trash]).reshape(nbt, EDGE_BATCH)
    w_padded = jnp.pad(W, ((0, 0), (0, cp - c_out)))
    b2d = b.reshape(1, c_out)

    y0 = _tc_project(feature, w_padded, n, cp)
    degs = _sc_degree(src2d, dst2d, n)
    u0, rio, rin = _tc_prep(y0, degs, n, cp)
    p1, p2, p3 = _sc_mega(u0, rio, src2d, dst2d, n, cp)
    return _tc_final(p1, p2, p3, y0, rin, b2d, n, c_out)


# speedup vs baseline: 1.3299x; 1.3299x over previous
import functools

import jax
import jax.numpy as jnp
from jax import lax
from jax.experimental import pallas as pl
from jax.experimental.pallas import tpu as pltpu
from jax.experimental.pallas import tpu_sc as plsc

NUM_CORES = 2
NUM_SUBCORES = 16
NUM_WORKERS = NUM_CORES * NUM_SUBCORES
EDGE_BATCH = 128

_SC_PARAMS = pltpu.CompilerParams(use_tc_tiling_on_sc=False)


def _row_chunks(n):
    for nz in range(NUM_SUBCORES, 0, -1):
        if n % nz == 0 and (n // nz) % 8 == 0:
            return nz, n // nz
    return 1, n


def _sc_degree(src2d, dst2d, n):
    nbt, b = src2d.shape
    nbw = nbt // NUM_WORKERS
    nz, zch = _row_chunks(n)
    mesh = plsc.VectorSubcoreMesh(core_axis_name="c", subcore_axis_name="s")

    @functools.partial(
        pl.kernel,
        out_type=jax.ShapeDtypeStruct((2, NUM_CORES, n, 16), jnp.float32),
        mesh=mesh,
        scratch_types=[
            pltpu.VMEM((nbw, b), jnp.int32),
            pltpu.VMEM((nbw, b), jnp.int32),
            pltpu.VMEM((b, 16), jnp.float32),
            pltpu.VMEM((zch, 16), jnp.float32),
            pltpu.VMEM_SHARED((n + 8, 16), jnp.float32),
            pltpu.VMEM_SHARED((n + 8, 16), jnp.float32),
            pltpu.SemaphoreType.DMA,
        ],
        compiler_params=_SC_PARAMS,
    )
    def deg_kernel(src_hbm, dst_hbm, out_hbm,
                   sidx, didx, ones_v, zbuf, dsrc_sh, ddst_sh, sem):
        c = lax.axis_index("c")
        s = lax.axis_index("s")
        wid = c * NUM_SUBCORES + s
        r0 = s * zch
        pltpu.sync_copy(src_hbm.at[pl.ds(wid * nbw, nbw)], sidx)
        pltpu.sync_copy(dst_hbm.at[pl.ds(wid * nbw, nbw)], didx)

        @pl.loop(0, b)
        def _(i):
            ones_v[i] = jnp.full((16,), 1.0, jnp.float32)

        @pl.when(s < nz)
        def _():
            @pl.loop(0, zch)
            def _(i):
                zbuf[i] = jnp.zeros((16,), jnp.float32)

            pltpu.sync_copy(zbuf, dsrc_sh.at[pl.ds(r0, zch)])
            pltpu.sync_copy(zbuf, ddst_sh.at[pl.ds(r0, zch)])

        plsc.subcore_barrier()

        @pl.loop(0, nbw)
        def _(i):
            pltpu.async_copy(ones_v, dsrc_sh.at[sidx.at[i]], sem, add=True)
            pltpu.async_copy(ones_v, ddst_sh.at[didx.at[i]], sem, add=True)

        @pl.loop(0, 2 * nbw)
        def _(i):
            pltpu.make_async_copy(ones_v, dsrc_sh.at[sidx.at[0]], sem).wait()

        plsc.subcore_barrier()

        @pl.when(s < nz)
        def _():
            pltpu.sync_copy(dsrc_sh.at[pl.ds(r0, zch)],
                            out_hbm.at[0, c, pl.ds(r0, zch)])
            pltpu.sync_copy(ddst_sh.at[pl.ds(r0, zch)],
                            out_hbm.at[1, c, pl.ds(r0, zch)])

    return deg_kernel(src2d, dst2d)


def _sc_mega(u0, rio, src2d, dst2d, n, cp):
    nbt, b = src2d.shape
    nbw = nbt // NUM_WORKERS
    nz, zch = _row_chunks(n)
    mch = 200 if zch % 200 == 0 else zch
    ncnk = zch // mch
    mesh = plsc.VectorSubcoreMesh(core_axis_name="c", subcore_axis_name="s")
    pshape = jax.ShapeDtypeStruct((NUM_CORES, n, cp), jnp.float32)

    @functools.partial(
        pl.kernel,
        out_type=[pshape, pshape, pshape,
                  jax.ShapeDtypeStruct((NUM_CORES, n + 8, cp), jnp.float32)],
        mesh=mesh,
        scratch_types=[
            pltpu.VMEM((nbw, b), jnp.int32),
            pltpu.VMEM((nbw, b), jnp.int32),
            pltpu.VMEM((b, cp), jnp.float32),
            pltpu.VMEM((b, cp), jnp.float32),
            pltpu.VMEM((b, cp), jnp.float32),
            pltpu.VMEM((b, cp), jnp.float32),
            pltpu.VMEM((mch, cp), jnp.float32),
            pltpu.VMEM((mch, cp), jnp.float32),
            pltpu.VMEM((mch, cp), jnp.float32),
            pltpu.VMEM((mch, cp), jnp.float32),
            pltpu.VMEM((mch, cp), jnp.float32),
            pltpu.VMEM_SHARED((n + 8, cp), jnp.float32),
            pltpu.SemaphoreType.DMA,
            pltpu.SemaphoreType.DMA,
            pltpu.SemaphoreType.DMA,
            pltpu.SemaphoreType.DMA,
            pltpu.SemaphoreType.DMA,
            pltpu.SemaphoreType.DMA,
            pltpu.SemaphoreType.DMA,
            pltpu.SemaphoreType.DMA,
            pltpu.SemaphoreType.DMA,
            pltpu.SemaphoreType.DMA,
            pltpu.SemaphoreType.REGULAR,
        ],
        compiler_params=_SC_PARAMS,
    )
    def mega_kernel(u0_hbm, rio_hbm, src_hbm, dst_hbm,
                    p1_hbm, p2_hbm, p3_hbm, u_scr,
                    sidx, didx, r0b, r1b, r2b, r3b,
                    i0, i1, i2, uca, ucb, acc_sh,
                    g0, g1, g2, g3, s0, s1, s2, s3, isem, osem, xsem):
        rows = (r0b, r1b, r2b, r3b)
        gsem = (g0, g1, g2, g3)
        ssem = (s0, s1, s2, s3)
        ucnks = (uca, ucb)
        c = lax.axis_index("c")
        s = lax.axis_index("s")
        wid = c * NUM_SUBCORES + s
        r0z = s * zch

        pltpu.sync_copy(src_hbm.at[pl.ds(wid * nbw, nbw)], sidx)
        pltpu.sync_copy(dst_hbm.at[pl.ds(wid * nbw, nbw)], didx)

        def zero_acc():
            @pl.when(s < nz)
            def _():
                @pl.loop(0, mch)
                def _(i):
                    for q in range(cp // 16):
                        i0[i, pl.ds(q * 16, 16)] = jnp.zeros(
                            (16,), jnp.float32)

                for t in range(ncnk):
                    pltpu.sync_copy(i0, acc_sh.at[pl.ds(r0z + t * mch, mch)])

        def sparse_phase(u_ref):
            @pl.loop(0, nbw + 4, step=4)
            def _(v):
                for j in range(4):
                    i = v + j

                    @pl.when((i >= 4) & (i < nbw))
                    def _():
                        pltpu.make_async_copy(
                            rows[j], acc_sh.at[didx.at[0]], ssem[j]).wait()

                    @pl.when(i < nbw)
                    def _():
                        pltpu.async_copy(u_ref.at[sidx.at[i]], rows[j],
                                         gsem[j])

                    k2 = i - 2
                    jb = (j + 2) % 4

                    @pl.when((k2 >= 0) & (k2 < nbw))
                    def _():
                        pltpu.make_async_copy(
                            u_ref.at[sidx.at[0]], rows[jb], gsem[jb]).wait()
                        pltpu.async_copy(rows[jb], acc_sh.at[didx.at[k2]],
                                         ssem[jb], add=True)

            for j in range(4):
                pltpu.make_async_copy(rows[j], acc_sh.at[didx.at[0]],
                                      ssem[j]).wait()

        def writeback(p_hbm):
            @pl.when(s < nz)
            def _():
                pltpu.sync_copy(acc_sh.at[pl.ds(r0z, zch)],
                                p_hbm.at[c, pl.ds(r0z, zch)])

        def xsync():
            plsc.subcore_barrier()
            pl.semaphore_signal(xsem, 1, core_index=1 - c)
            pl.semaphore_wait(xsem, 1)

        def combine(p_ref):
            @pl.when(s < nz)
            def _():
                def fetch(t):
                    rr = r0z + t * mch
                    pltpu.async_copy(p_ref.at[0, pl.ds(rr, mch)], i0, isem)
                    pltpu.async_copy(p_ref.at[1, pl.ds(rr, mch)], i1, isem)
                    pltpu.async_copy(rio_hbm.at[pl.ds(rr, mch)], i2, isem)

                fetch(0)
                for t in range(ncnk):
                    rr = r0z + t * mch
                    uc = ucnks[t % 2]
                    for _ in range(3):
                        pltpu.make_async_copy(
                            rio_hbm.at[pl.ds(rr, mch)], i2, isem).wait()
                    if t >= 2:
                        pltpu.make_async_copy(
                            uc, u_scr.at[c, pl.ds(rr, mch)], osem).wait()

                    @pl.loop(0, mch, step=8)
                    def _(r):
                        for dr in range(8):
                            for q in range(cp // 16):
                                sl = (r + dr, pl.ds(q * 16, 16))
                                uc[sl] = i2[sl] * (i0[sl] + i1[sl])

                    if t + 1 < ncnk:
                        fetch(t + 1)
                    pltpu.async_copy(uc, u_scr.at[c, pl.ds(rr, mch)], osem)

                for _ in range(min(2, ncnk)):
                    pltpu.make_async_copy(
                        uca, u_scr.at[c, pl.ds(r0z, mch)], osem).wait()

        zero_acc()

        @pl.when(s == 0)
        def _():
            pltpu.sync_copy(i0.at[pl.ds(0, 8)], u_scr.at[c, pl.ds(n, 8)])

        plsc.subcore_barrier()
        sparse_phase(u0_hbm)
        plsc.subcore_barrier()
        writeback(p1_hbm)
        zero_acc()
        xsync()

        combine(p1_hbm)
        plsc.subcore_barrier()
        sparse_phase(u_scr.at[c])
        plsc.subcore_barrier()
        writeback(p2_hbm)
        zero_acc()
        xsync()

        combine(p2_hbm)
        plsc.subcore_barrier()
        sparse_phase(u_scr.at[c])
        plsc.subcore_barrier()
        writeback(p3_hbm)

    res = mega_kernel(u0, rio, src2d, dst2d)
    return res[0], res[1], res[2]


def _tc_project(feature, w_padded, n, cp):
    blk = 1000

    def body(f_ref, w_ref, o_ref):
        o_ref[...] = jnp.dot(f_ref[...], w_ref[...],
                             preferred_element_type=jnp.float32)

    d = feature.shape[1]
    return pl.pallas_call(
        body,
        grid=(n // blk,),
        in_specs=[pl.BlockSpec((blk, d), lambda i: (i, 0)),
                  pl.BlockSpec((d, cp), lambda i: (0, 0))],
        out_specs=pl.BlockSpec((blk, cp), lambda i: (i, 0)),
        out_shape=jax.ShapeDtypeStruct((n, cp), jnp.float32),
    )(feature, w_padded)


def _tc_prep(y0, degs, n, cp):
    def body(y_ref, d_ref, u_ref, rio_ref, rin_ref):
        deg_out = d_ref[0, 0, :, 0:1] + d_ref[0, 1, :, 0:1]
        deg_in = d_ref[1, 0, :, 0:1] + d_ref[1, 1, :, 0:1]
        rout = lax.rsqrt(jnp.maximum(deg_out, 1.0))
        rin = lax.rsqrt(jnp.maximum(deg_in, 1.0))
        u_ref[...] = jnp.concatenate(
            [y_ref[...] * rout, jnp.zeros((8, cp), jnp.float32)], axis=0)
        rio_ref[...] = jnp.broadcast_to(rin * rout, (n, cp))
        rin_ref[...] = rin

    return pl.pallas_call(
        body,
        out_shape=[
            jax.ShapeDtypeStruct((n + 8, cp), jnp.float32),
            jax.ShapeDtypeStruct((n, cp), jnp.float32),
            jax.ShapeDtypeStruct((n, 1), jnp.float32),
        ],
    )(y0, degs)


def _tc_final(p1, p2, p3, y0, rin, b2d, n, c_out):
    def body(p1_ref, p2_ref, p3_ref, y_ref, rin_ref, b_ref, o_ref):
        psum = ((p1_ref[0] + p1_ref[1]) + (p2_ref[0] + p2_ref[1])
                + (p3_ref[0] + p3_ref[1]))
        res = (y_ref[...] + psum * rin_ref[...]) * 0.25
        o_ref[...] = res[:, :c_out] + b_ref[...]

    return pl.pallas_call(
        body, out_shape=jax.ShapeDtypeStruct((n, c_out), jnp.float32),
    )(p1, p2, p3, y0, rin, b2d)


def kernel(feature, edge_index, W, b):
    n, d = feature.shape
    c_out = W.shape[1]
    e = edge_index.shape[1]
    cp = ((c_out + 15) // 16) * 16

    nbt = -(-e // (EDGE_BATCH * NUM_WORKERS)) * NUM_WORKERS
    epad = nbt * EDGE_BATCH - e
    pad_idx = jnp.arange(epad, dtype=jnp.int32)
    trash8 = n + (pad_idx % 8)
    src2d = jnp.concatenate([edge_index[0], trash8]).reshape(nbt, EDGE_BATCH)
    dst_deg = jnp.concatenate([edge_index[1], trash8]).reshape(nbt, EDGE_BATCH)
    dst2d = jnp.concatenate([edge_index[1], pad_idx % jnp.int32(n)]
                            ).reshape(nbt, EDGE_BATCH)
    w_padded = jnp.pad(W, ((0, 0), (0, cp - c_out)))
    b2d = b.reshape(1, c_out)

    y0 = _tc_project(feature, w_padded, n, cp)
    degs = _sc_degree(src2d, dst_deg, n)
    u0, rio, rin = _tc_prep(y0, degs, n, cp)
    p1, p2, p3 = _sc_mega(u0, rio, src2d, dst_deg, n, cp)
    return _tc_final(p1, p2, p3, y0, rin, b2d, n, c_out)


# speedup vs baseline: 1.5527x; 1.1675x over previous
import functools

import jax
import jax.numpy as jnp
from jax import lax
from jax.experimental import pallas as pl
from jax.experimental.pallas import tpu as pltpu
from jax.experimental.pallas import tpu_sc as plsc

NUM_CORES = 2
NUM_SUBCORES = 16
NUM_WORKERS = NUM_CORES * NUM_SUBCORES


def _pick_batch(edges_per_worker):
    for cand in range(128, 0, -1):
        if edges_per_worker % cand == 0:
            return cand
    return 1

_SC_PARAMS = pltpu.CompilerParams(use_tc_tiling_on_sc=False)


def _row_chunks(n):
    for nz in range(NUM_SUBCORES, 0, -1):
        if n % nz == 0 and (n // nz) % 8 == 0:
            return nz, n // nz
    return 1, n


def _sc_degree(src2d, dst2d, n):
    nbt, b = src2d.shape
    nbw = nbt // NUM_WORKERS
    nz, zch = _row_chunks(n)
    mesh = plsc.VectorSubcoreMesh(core_axis_name="c", subcore_axis_name="s")

    @functools.partial(
        pl.kernel,
        out_type=jax.ShapeDtypeStruct((2, NUM_CORES, n, 16), jnp.float32),
        mesh=mesh,
        scratch_types=[
            pltpu.VMEM((nbw, b), jnp.int32),
            pltpu.VMEM((nbw, b), jnp.int32),
            pltpu.VMEM((b, 16), jnp.float32),
            pltpu.VMEM((zch, 16), jnp.float32),
            pltpu.VMEM_SHARED((n + 8, 16), jnp.float32),
            pltpu.VMEM_SHARED((n + 8, 16), jnp.float32),
            pltpu.SemaphoreType.DMA,
        ],
        compiler_params=_SC_PARAMS,
    )
    def deg_kernel(src_hbm, dst_hbm, out_hbm,
                   sidx, didx, ones_v, zbuf, dsrc_sh, ddst_sh, sem):
        c = lax.axis_index("c")
        s = lax.axis_index("s")
        wid = c * NUM_SUBCORES + s
        r0 = s * zch
        pltpu.sync_copy(src_hbm.at[pl.ds(wid * nbw, nbw)], sidx)
        pltpu.sync_copy(dst_hbm.at[pl.ds(wid * nbw, nbw)], didx)

        @pl.loop(0, b)
        def _(i):
            ones_v[i] = jnp.full((16,), 1.0, jnp.float32)

        @pl.when(s < nz)
        def _():
            @pl.loop(0, zch)
            def _(i):
                zbuf[i] = jnp.zeros((16,), jnp.float32)

            pltpu.sync_copy(zbuf, dsrc_sh.at[pl.ds(r0, zch)])
            pltpu.sync_copy(zbuf, ddst_sh.at[pl.ds(r0, zch)])

        plsc.subcore_barrier()

        @pl.loop(0, nbw)
        def _(i):
            pltpu.async_copy(ones_v, dsrc_sh.at[sidx.at[i]], sem, add=True)
            pltpu.async_copy(ones_v, ddst_sh.at[didx.at[i]], sem, add=True)

        @pl.loop(0, 2 * nbw)
        def _(i):
            pltpu.make_async_copy(ones_v, dsrc_sh.at[sidx.at[0]], sem).wait()

        plsc.subcore_barrier()

        @pl.when(s < nz)
        def _():
            pltpu.sync_copy(dsrc_sh.at[pl.ds(r0, zch)],
                            out_hbm.at[0, c, pl.ds(r0, zch)])
            pltpu.sync_copy(ddst_sh.at[pl.ds(r0, zch)],
                            out_hbm.at[1, c, pl.ds(r0, zch)])

    return deg_kernel(src2d, dst2d)


def _sc_mega(u0, rio, src2d, dst2d, n, cp):
    nbt, b = src2d.shape
    nbw = nbt // NUM_WORKERS
    nz, zch = _row_chunks(n)
    mch = 200 if zch % 200 == 0 else zch
    ncnk = zch // mch
    mesh = plsc.VectorSubcoreMesh(core_axis_name="c", subcore_axis_name="s")
    pshape = jax.ShapeDtypeStruct((NUM_CORES, n, cp), jnp.float32)

    @functools.partial(
        pl.kernel,
        out_type=[pshape, pshape, pshape,
                  jax.ShapeDtypeStruct((NUM_CORES, n + 8, cp), jnp.float32)],
        mesh=mesh,
        scratch_types=[
            pltpu.VMEM((nbw, b), jnp.int32),
            pltpu.VMEM((nbw, b), jnp.int32),
            pltpu.VMEM((b, cp), jnp.float32),
            pltpu.VMEM((b, cp), jnp.float32),
            pltpu.VMEM((b, cp), jnp.float32),
            pltpu.VMEM((b, cp), jnp.float32),
            pltpu.VMEM((mch, cp), jnp.float32),
            pltpu.VMEM((mch, cp), jnp.float32),
            pltpu.VMEM((mch, cp), jnp.float32),
            pltpu.VMEM((mch, cp), jnp.float32),
            pltpu.VMEM((mch, cp), jnp.float32),
            pltpu.VMEM_SHARED((n + 8, cp), jnp.float32),
            pltpu.SemaphoreType.DMA,
            pltpu.SemaphoreType.DMA,
            pltpu.SemaphoreType.DMA,
            pltpu.SemaphoreType.DMA,
            pltpu.SemaphoreType.DMA,
            pltpu.SemaphoreType.DMA,
            pltpu.SemaphoreType.DMA,
            pltpu.SemaphoreType.DMA,
            pltpu.SemaphoreType.DMA,
            pltpu.SemaphoreType.DMA,
            pltpu.SemaphoreType.REGULAR,
        ],
        compiler_params=_SC_PARAMS,
    )
    def mega_kernel(u0_hbm, rio_hbm, src_hbm, dst_hbm,
                    p1_hbm, p2_hbm, p3_hbm, u_scr,
                    sidx, didx, r0b, r1b, r2b, r3b,
                    i0, i1, i2, uca, ucb, acc_sh,
                    g0, g1, g2, g3, s0, s1, s2, s3, isem, osem, xsem):
        rows = (r0b, r1b, r2b, r3b)
        gsem = (g0, g1, g2, g3)
        ssem = (s0, s1, s2, s3)
        ucnks = (uca, ucb)
        c = lax.axis_index("c")
        s = lax.axis_index("s")
        wid = c * NUM_SUBCORES + s
        r0z = s * zch

        pltpu.sync_copy(src_hbm.at[pl.ds(wid * nbw, nbw)], sidx)
        pltpu.sync_copy(dst_hbm.at[pl.ds(wid * nbw, nbw)], didx)

        def zero_acc():
            @pl.when(s < nz)
            def _():
                @pl.loop(0, mch)
                def _(i):
                    for q in range(cp // 16):
                        i0[i, pl.ds(q * 16, 16)] = jnp.zeros(
                            (16,), jnp.float32)

                for t in range(ncnk):
                    pltpu.sync_copy(i0, acc_sh.at[pl.ds(r0z + t * mch, mch)])

        def sparse_phase(u_ref):
            @pl.loop(0, nbw + 4, step=4)
            def _(v):
                for j in range(4):
                    i = v + j

                    @pl.when((i >= 4) & (i < nbw))
                    def _():
                        pltpu.make_async_copy(
                            rows[j], acc_sh.at[didx.at[0]], ssem[j]).wait()

                    @pl.when(i < nbw)
                    def _():
                        pltpu.async_copy(u_ref.at[sidx.at[i]], rows[j],
                                         gsem[j])

                    k2 = i - 2
                    jb = (j + 2) % 4

                    @pl.when((k2 >= 0) & (k2 < nbw))
                    def _():
                        pltpu.make_async_copy(
                            u_ref.at[sidx.at[0]], rows[jb], gsem[jb]).wait()
                        pltpu.async_copy(rows[jb], acc_sh.at[didx.at[k2]],
                                         ssem[jb], add=True)

            for j in range(4):
                pltpu.make_async_copy(rows[j], acc_sh.at[didx.at[0]],
                                      ssem[j]).wait()

        def writeback(p_hbm):
            @pl.when(s < nz)
            def _():
                pltpu.sync_copy(acc_sh.at[pl.ds(r0z, zch)],
                                p_hbm.at[c, pl.ds(r0z, zch)])

        def xsync():
            plsc.subcore_barrier()
            pl.semaphore_signal(xsem, 1, core_index=1 - c)
            pl.semaphore_wait(xsem, 1)

        def combine(p_ref):
            @pl.when(s < nz)
            def _():
                def fetch(t):
                    rr = r0z + t * mch
                    pltpu.async_copy(p_ref.at[0, pl.ds(rr, mch)], i0, isem)
                    pltpu.async_copy(p_ref.at[1, pl.ds(rr, mch)], i1, isem)
                    pltpu.async_copy(rio_hbm.at[pl.ds(rr, mch)], i2, isem)

                fetch(0)
                for t in range(ncnk):
                    rr = r0z + t * mch
                    uc = ucnks[t % 2]
                    for _ in range(3):
                        pltpu.make_async_copy(
                            rio_hbm.at[pl.ds(rr, mch)], i2, isem).wait()
                    if t >= 2:
                        pltpu.make_async_copy(
                            uc, u_scr.at[c, pl.ds(rr, mch)], osem).wait()

                    @pl.loop(0, mch, step=8)
                    def _(r):
                        for dr in range(8):
                            for q in range(cp // 16):
                                sl = (r + dr, pl.ds(q * 16, 16))
                                uc[sl] = i2[sl] * (i0[sl] + i1[sl])

                    if t + 1 < ncnk:
                        fetch(t + 1)
                    pltpu.async_copy(uc, u_scr.at[c, pl.ds(rr, mch)], osem)

                for _ in range(min(2, ncnk)):
                    pltpu.make_async_copy(
                        uca, u_scr.at[c, pl.ds(r0z, mch)], osem).wait()

        zero_acc()

        @pl.when(s == 0)
        def _():
            pltpu.sync_copy(i0.at[pl.ds(0, 8)], u_scr.at[c, pl.ds(n, 8)])

        plsc.subcore_barrier()
        sparse_phase(u0_hbm)
        plsc.subcore_barrier()
        writeback(p1_hbm)
        zero_acc()
        xsync()

        combine(p1_hbm)
        plsc.subcore_barrier()
        sparse_phase(u_scr.at[c])
        plsc.subcore_barrier()
        writeback(p2_hbm)
        zero_acc()
        xsync()

        combine(p2_hbm)
        plsc.subcore_barrier()
        sparse_phase(u_scr.at[c])
        plsc.subcore_barrier()
        writeback(p3_hbm)

    res = mega_kernel(u0, rio, src2d, dst2d)
    return res[0], res[1], res[2]


def _tc_project(feature, w_padded, n, cp):
    blk = 1000

    def body(f_ref, w_ref, o_ref):
        o_ref[...] = jnp.dot(f_ref[...], w_ref[...],
                             preferred_element_type=jnp.float32)

    d = feature.shape[1]
    return pl.pallas_call(
        body,
        grid=(n // blk,),
        in_specs=[pl.BlockSpec((blk, d), lambda i: (i, 0)),
                  pl.BlockSpec((d, cp), lambda i: (0, 0))],
        out_specs=pl.BlockSpec((blk, cp), lambda i: (i, 0)),
        out_shape=jax.ShapeDtypeStruct((n, cp), jnp.float32),
    )(feature, w_padded)


def _tc_prep(y0, degs, n, cp):
    def body(y_ref, d_ref, u_ref, rio_ref, rin_ref):
        deg_out = d_ref[0, 0, :, 0:1] + d_ref[0, 1, :, 0:1]
        deg_in = d_ref[1, 0, :, 0:1] + d_ref[1, 1, :, 0:1]
        rout = lax.rsqrt(jnp.maximum(deg_out, 1.0))
        rin = lax.rsqrt(jnp.maximum(deg_in, 1.0))
        u_ref[...] = jnp.concatenate(
            [y_ref[...] * rout, jnp.zeros((8, cp), jnp.float32)], axis=0)
        rio_ref[...] = jnp.broadcast_to(rin * rout, (n, cp))
        rin_ref[...] = rin

    return pl.pallas_call(
        body,
        out_shape=[
            jax.ShapeDtypeStruct((n + 8, cp), jnp.float32),
            jax.ShapeDtypeStruct((n, cp), jnp.float32),
            jax.ShapeDtypeStruct((n, 1), jnp.float32),
        ],
    )(y0, degs)


def _tc_final(p1, p2, p3, y0, rin, b2d, n, c_out):
    def body(p1_ref, p2_ref, p3_ref, y_ref, rin_ref, b_ref, o_ref):
        psum = ((p1_ref[0] + p1_ref[1]) + (p2_ref[0] + p2_ref[1])
                + (p3_ref[0] + p3_ref[1]))
        res = (y_ref[...] + psum * rin_ref[...]) * 0.25
        o_ref[...] = res[:, :c_out] + b_ref[...]

    return pl.pallas_call(
        body, out_shape=jax.ShapeDtypeStruct((n, c_out), jnp.float32),
    )(p1, p2, p3, y0, rin, b2d)


def kernel(feature, edge_index, W, b):
    n, d = feature.shape
    c_out = W.shape[1]
    e = edge_index.shape[1]
    cp = ((c_out + 15) // 16) * 16

    eb = _pick_batch(e // NUM_WORKERS)
    nbt = e // eb
    src2d = edge_index[0].reshape(nbt, eb)
    dst2d = edge_index[1].reshape(nbt, eb)
    w_padded = jnp.pad(W, ((0, 0), (0, cp - c_out)))
    b2d = b.reshape(1, c_out)

    y0 = _tc_project(feature, w_padded, n, cp)
    degs = _sc_degree(src2d, dst2d, n)
    u0, rio, rin = _tc_prep(y0, degs, n, cp)
    p1, p2, p3 = _sc_mega(u0, rio, src2d, dst2d, n, cp)
    return _tc_final(p1, p2, p3, y0, rin, b2d, n, c_out)
